# baseline (speedup 1.0000x reference)
import functools

import jax
import jax.numpy as jnp
import numpy as np
from jax import lax
from jax.experimental import pallas as pl
from jax.experimental.pallas import tpu as pltpu
from jax.experimental.pallas import tpu_sc as plsc

NUM_FIELDS = 26
FIELD_DIM = 38462
BATCH = 16384
EMBED_DIM = 16
_OFFSETS = (np.arange(NUM_FIELDS, dtype=np.int32) * FIELD_DIM)

NC = 2
NS = 16
NW = NC * NS
BPW = BATCH // NW
CHUNK = 64
NCHUNK = BPW // CHUNK
ROWS_PER_CHUNK = CHUNK * NUM_FIELDS


def _fm_body(xo_hbm, w_hbm, wlin_hbm, blin_hbm, out_hbm,
             idxv0, idxv1, rowsv0, rowsv1, linv0, linv1, tbuf, outv, bv,
             sem_w0, sem_w1, sem_l0, sem_l1):
    wid = lax.axis_index("s") * NC + lax.axis_index("c")
    pltpu.sync_copy(blin_hbm, bv)
    bvec = bv[...]
    iota = lax.iota(jnp.int32, 16)
    zero16 = jnp.zeros((16,), jnp.int32)

    idxv = (idxv0, idxv1)
    rowsv = (rowsv0, rowsv1)
    linv = (linv0, linv1)
    sem_w = (sem_w0, sem_w1)
    sem_l = (sem_l0, sem_l1)
    base = wid * BPW * NUM_FIELDS

    def fire(c, slot):
        pltpu.sync_copy(
            xo_hbm.at[pl.ds(base + c * ROWS_PER_CHUNK, ROWS_PER_CHUNK)],
            idxv[slot])
        cw = pltpu.async_copy(w_hbm.at[idxv[slot]], rowsv[slot], sem_w[slot])
        cl = pltpu.async_copy(wlin_hbm.at[idxv[slot]], linv[slot], sem_l[slot])
        return cw, cl

    inflight = fire(0, 0)
    for c in range(NCHUNK):
        slot = c % 2
        cw, cl = inflight
        if c + 1 < NCHUNK:
            inflight = fire(c + 1, 1 - slot)
        cw.wait()
        cl.wait()
        rv = rowsv[slot]
        lv = linv[slot]

        def bi_body(bi, _):
            p0 = bi * NUM_FIELDS
            acc_s = jnp.zeros((16,), jnp.float32)
            acc_q = jnp.zeros((16,), jnp.float32)
            for f in range(NUM_FIELDS):
                v = rv[p0 + f]
                acc_s = acc_s + v
                acc_q = acc_q + v * v
            tbuf[pl.ds(bi * 16, 16)] = acc_s * acc_s - acc_q
            return 0

        lax.fori_loop(0, CHUNK, bi_body, 0, unroll=False)

        def q_body(q, _):
            bi_v = q * 16 + iota
            ti0 = bi_v * 16
            fm = jnp.zeros((16,), jnp.float32)
            for j in range(16):
                fm = fm + plsc.load_gather(tbuf, [ti0 + j])
            lin = jnp.zeros((16,), jnp.float32)
            pv0 = bi_v * NUM_FIELDS
            for f in range(NUM_FIELDS):
                lin = lin + plsc.load_gather(lv, [pv0 + f, zero16])
            outv[pl.ds(c * CHUNK + q * 16, 16)] = lin + 0.5 * fm + bvec
            return 0

        lax.fori_loop(0, CHUNK // 16, q_body, 0, unroll=False)

    pltpu.sync_copy(outv, out_hbm.at[pl.ds(wid * BPW, BPW)])


@jax.jit
def kernel(x, W, w_lin, b_lin):
    xo = (x + jnp.asarray(_OFFSETS)[None, :]).reshape(-1)
    blin16 = jnp.broadcast_to(b_lin.astype(jnp.float32), (16,))

    mesh = plsc.VectorSubcoreMesh(core_axis_name="c", subcore_axis_name="s")
    fm_kernel = pl.kernel(
        _fm_body,
        out_type=jax.ShapeDtypeStruct((BATCH,), jnp.float32),
        mesh=mesh,
        compiler_params=pltpu.CompilerParams(
            needs_layout_passes=False, use_tc_tiling_on_sc=False),
        scratch_types=[
            pltpu.VMEM((ROWS_PER_CHUNK,), jnp.int32),
            pltpu.VMEM((ROWS_PER_CHUNK,), jnp.int32),
            pltpu.VMEM((ROWS_PER_CHUNK, EMBED_DIM), jnp.float32),
            pltpu.VMEM((ROWS_PER_CHUNK, EMBED_DIM), jnp.float32),
            pltpu.VMEM((ROWS_PER_CHUNK, 1), jnp.float32),
            pltpu.VMEM((ROWS_PER_CHUNK, 1), jnp.float32),
            pltpu.VMEM((CHUNK * 16,), jnp.float32),
            pltpu.VMEM((BPW,), jnp.float32),
            pltpu.VMEM((16,), jnp.float32),
            pltpu.SemaphoreType.DMA,
            pltpu.SemaphoreType.DMA,
            pltpu.SemaphoreType.DMA,
            pltpu.SemaphoreType.DMA,
        ],
    )
    return fm_kernel(xo, W, w_lin, blin16)


# speedup vs baseline: 2.5405x; 2.5405x over previous
import functools

import jax
import jax.numpy as jnp
import numpy as np
from jax import lax
from jax.experimental import pallas as pl
from jax.experimental.pallas import tpu as pltpu
from jax.experimental.pallas import tpu_sc as plsc

NUM_FIELDS = 26
FIELD_DIM = 38462
BATCH = 16384
EMBED_DIM = 16
_OFFSETS = (np.arange(NUM_FIELDS, dtype=np.int32) * FIELD_DIM)

NC = 2
NS = 16
NW = NC * NS
BPW = BATCH // NW
CHUNK = 64
NCHUNK = BPW // CHUNK
ROWS_PER_CHUNK = CHUNK * NUM_FIELDS


def _fm_body(xo_hbm, w_hbm, wlin_hbm, blin_hbm, out_hbm,
             idxv0, idxv1, rowsv0, rowsv1, linv0, linv1, tbuf, outv, bv,
             sem_w0, sem_w1, sem_l0, sem_l1):
    wid = lax.axis_index("s") * NC + lax.axis_index("c")
    pltpu.sync_copy(blin_hbm, bv)
    bvec = bv[...]
    iota = lax.iota(jnp.int32, 16)
    zero16 = jnp.zeros((16,), jnp.int32)

    idxv = (idxv0, idxv1)
    rowsv = (rowsv0, rowsv1)
    linv = (linv0, linv1)
    sem_w = (sem_w0, sem_w1)
    sem_l = (sem_l0, sem_l1)
    base = wid * BPW * NUM_FIELDS

    def fire(c, slot):
        pltpu.sync_copy(
            xo_hbm.at[pl.ds(base + c * ROWS_PER_CHUNK, ROWS_PER_CHUNK)],
            idxv[slot])
        cw = pltpu.async_copy(w_hbm.at[idxv[slot]], rowsv[slot], sem_w[slot])
        cl = pltpu.async_copy(wlin_hbm.at[idxv[slot]], linv[slot], sem_l[slot])
        return cw, cl

    inflight = fire(0, 0)
    for c in range(NCHUNK):
        slot = c % 2
        cw, cl = inflight
        if c + 1 < NCHUNK:
            inflight = fire(c + 1, 1 - slot)
        cw.wait()
        cl.wait()
        rv = rowsv[slot]
        lv = linv[slot]

        def bi_body(bi, _):
            p0 = bi * NUM_FIELDS
            acc_s = jnp.zeros((16,), jnp.float32)
            acc_q = jnp.zeros((16,), jnp.float32)
            for f in range(NUM_FIELDS):
                v = rv[p0 + f]
                acc_s = acc_s + v
                acc_q = acc_q + v * v
            tbuf[pl.ds(bi * 16, 16)] = acc_s * acc_s - acc_q
            return 0

        lax.fori_loop(0, CHUNK, bi_body, 0, unroll=False)

        def q_body(q, _):
            bi_v = q * 16 + iota
            ti0 = bi_v * 16
            fm = jnp.zeros((16,), jnp.float32)
            for j in range(16):
                fm = fm + plsc.load_gather(tbuf, [ti0 + j])
            lin = jnp.zeros((16,), jnp.float32)
            pv0 = bi_v * NUM_FIELDS
            for f in range(NUM_FIELDS):
                lin = lin + plsc.load_gather(lv, [pv0 + f])
            outv[pl.ds(c * CHUNK + q * 16, 16)] = lin + 0.5 * fm + bvec
            return 0

        lax.fori_loop(0, CHUNK // 16, q_body, 0, unroll=False)

    pltpu.sync_copy(outv, out_hbm.at[pl.ds(wid * BPW, BPW)])


@jax.jit
def kernel(x, W, w_lin, b_lin):
    xo = (x + jnp.asarray(_OFFSETS)[None, :]).reshape(-1)
    blin16 = jnp.broadcast_to(b_lin.astype(jnp.float32), (16,))

    mesh = plsc.VectorSubcoreMesh(core_axis_name="c", subcore_axis_name="s")
    fm_kernel = pl.kernel(
        _fm_body,
        out_type=jax.ShapeDtypeStruct((BATCH,), jnp.float32),
        mesh=mesh,
        compiler_params=pltpu.CompilerParams(
            needs_layout_passes=False, use_tc_tiling_on_sc=False),
        scratch_types=[
            pltpu.VMEM((ROWS_PER_CHUNK,), jnp.int32),
            pltpu.VMEM((ROWS_PER_CHUNK,), jnp.int32),
            pltpu.VMEM((ROWS_PER_CHUNK, EMBED_DIM), jnp.float32),
            pltpu.VMEM((ROWS_PER_CHUNK, EMBED_DIM), jnp.float32),
            pltpu.VMEM((ROWS_PER_CHUNK,), jnp.float32),
            pltpu.VMEM((ROWS_PER_CHUNK,), jnp.float32),
            pltpu.VMEM((CHUNK * 16,), jnp.float32),
            pltpu.VMEM((BPW,), jnp.float32),
            pltpu.VMEM((16,), jnp.float32),
            pltpu.SemaphoreType.DMA,
            pltpu.SemaphoreType.DMA,
            pltpu.SemaphoreType.DMA,
            pltpu.SemaphoreType.DMA,
        ],
    )
    return fm_kernel(xo, W, w_lin.reshape(-1), blin16)
